# initial kernel scaffold (unmeasured)
import jax
import jax.numpy as jnp
from jax import lax
from jax.experimental import pallas as pl
from jax.experimental.pallas import tpu as pltpu

N = 4
T = 1024
D = 512


def kernel(x, dest):
    dest2 = dest.reshape(1, T)

    def body(x_ref, d_ref, out_ref, xg, dg, sx, sd, rx, rd):
        mx = lax.axis_index("x")
        my = lax.axis_index("y")
        mz = lax.axis_index("z")

        bsem = pltpu.get_barrier_semaphore()
        for dy in range(1, N):
            peer = lax.rem(my + dy, N)
            pl.semaphore_signal(
                bsem, inc=1,
                device_id=(mx, peer, mz),
                device_id_type=pl.DeviceIdType.MESH,
            )
        pl.semaphore_wait(bsem, N - 1)

        dg[pl.ds(my, 1), :] = d_ref[:, :]
        xg[pl.ds(my, 1), :, :] = x_ref[:, :].astype(jnp.bfloat16).reshape(1, T, D)

        for dy in range(1, N):
            peer = lax.rem(my + dy, N)
            pltpu.make_async_remote_copy(
                src_ref=dg.at[pl.ds(my, 1)],
                dst_ref=dg.at[pl.ds(my, 1)],
                send_sem=sd, recv_sem=rd,
                device_id=(mx, peer, mz),
                device_id_type=pl.DeviceIdType.MESH,
            ).start()
            pltpu.make_async_remote_copy(
                src_ref=xg.at[pl.ds(my, 1)],
                dst_ref=xg.at[pl.ds(my, 1)],
                send_sem=sx, recv_sem=rx,
                device_id=(mx, peer, mz),
                device_id_type=pl.DeviceIdType.MESH,
            ).start()

        for _ in range(N - 1):
            dummy_d = pltpu.make_async_remote_copy(
                src_ref=dg.at[pl.ds(0, 1)], dst_ref=dg.at[pl.ds(0, 1)],
                send_sem=sd, recv_sem=rd,
                device_id=(mx, my, mz),
                device_id_type=pl.DeviceIdType.MESH,
            )
            dummy_d.wait_send()
            dummy_d.wait_recv()
            dummy_x = pltpu.make_async_remote_copy(
                src_ref=xg.at[pl.ds(0, 1)], dst_ref=xg.at[pl.ds(0, 1)],
                send_sem=sx, recv_sem=rx,
                device_id=(mx, my, mz),
                device_id_type=pl.DeviceIdType.MESH,
            )
            dummy_x.wait_send()
            dummy_x.wait_recv()

        ii = lax.broadcasted_iota(jnp.int32, (T, T), 0)
        jj = lax.broadcasted_iota(jnp.int32, (T, T), 1)
        upper = (ii <= jj).astype(jnp.float32)
        piota = lax.broadcasted_iota(jnp.float32, (T, T), 0)

        acc = jnp.zeros((T, D), jnp.float32)
        total = jnp.float32(0.0)
        for s in range(N):
            mrow = (dg[pl.ds(s, 1), :] == my).astype(jnp.float32)
            csum = jnp.dot(mrow, upper, preferred_element_type=jnp.float32)
            pos = total + csum - 1.0
            sel = ((piota == pos) & (mrow > 0.5)).astype(jnp.bfloat16)
            acc = acc + jnp.dot(sel, xg[s], preferred_element_type=jnp.float32)
            total = total + jnp.sum(mrow)
        out_ref[:, :] = acc.astype(jnp.bfloat16)

    return pl.pallas_call(
        body,
        out_shape=jax.ShapeDtypeStruct((T, D), jnp.bfloat16),
        in_specs=[
            pl.BlockSpec(memory_space=pltpu.VMEM),
            pl.BlockSpec(memory_space=pltpu.VMEM),
        ],
        out_specs=pl.BlockSpec(memory_space=pltpu.VMEM),
        scratch_shapes=[
            pltpu.VMEM((N, T, D), jnp.bfloat16),
            pltpu.VMEM((N, T), jnp.int32),
            pltpu.SemaphoreType.DMA,
            pltpu.SemaphoreType.DMA,
            pltpu.SemaphoreType.DMA,
            pltpu.SemaphoreType.DMA,
        ],
        compiler_params=pltpu.CompilerParams(collective_id=0),
    )(x, dest2)


# baseline (device time: 58425 ns/iter reference)
import jax
import jax.numpy as jnp
from jax import lax
from jax.experimental import pallas as pl
from jax.experimental.pallas import tpu as pltpu

N = 4
T = 1024
D = 512


def kernel(x, dest):
    dest2 = dest.reshape(1, T)

    def body(x_ref, d_ref, out_ref, xg, dg, sx, sd, rx, rd):
        mx = lax.axis_index("x")
        my = lax.axis_index("y")
        mz = lax.axis_index("z")

        bsem = pltpu.get_barrier_semaphore()
        for dy in range(1, N):
            peer = lax.rem(my + dy, N)
            pl.semaphore_signal(
                bsem, inc=1,
                device_id=(mx, peer, mz),
                device_id_type=pl.DeviceIdType.MESH,
            )
        pl.semaphore_wait(bsem, N - 1)

        dg[pl.ds(my, 1), :] = d_ref[:, :]
        xg[pl.ds(my, 1), :, :] = x_ref[:, :].astype(jnp.bfloat16).reshape(1, T, D)

        for dy in range(1, N):
            peer = lax.rem(my + dy, N)
            pltpu.make_async_remote_copy(
                src_ref=dg.at[pl.ds(my, 1)],
                dst_ref=dg.at[pl.ds(my, 1)],
                send_sem=sd, recv_sem=rd,
                device_id=(mx, peer, mz),
                device_id_type=pl.DeviceIdType.MESH,
            ).start()
            pltpu.make_async_remote_copy(
                src_ref=xg.at[pl.ds(my, 1)],
                dst_ref=xg.at[pl.ds(my, 1)],
                send_sem=sx, recv_sem=rx,
                device_id=(mx, peer, mz),
                device_id_type=pl.DeviceIdType.MESH,
            ).start()

        for _ in range(N - 1):
            dummy_d = pltpu.make_async_remote_copy(
                src_ref=dg.at[pl.ds(0, 1)], dst_ref=dg.at[pl.ds(0, 1)],
                send_sem=sd, recv_sem=rd,
                device_id=(mx, my, mz),
                device_id_type=pl.DeviceIdType.MESH,
            )
            dummy_d.wait_send()
            dummy_d.wait_recv()
            dummy_x = pltpu.make_async_remote_copy(
                src_ref=xg.at[pl.ds(0, 1)], dst_ref=xg.at[pl.ds(0, 1)],
                send_sem=sx, recv_sem=rx,
                device_id=(mx, my, mz),
                device_id_type=pl.DeviceIdType.MESH,
            )
            dummy_x.wait_send()
            dummy_x.wait_recv()

        ii = lax.broadcasted_iota(jnp.int32, (T, T), 0)
        jj = lax.broadcasted_iota(jnp.int32, (T, T), 1)
        upper = (ii <= jj).astype(jnp.float32)

        acc = jnp.zeros((T, D), jnp.float32)
        total = jnp.float32(0.0)
        for s in range(N):
            mrow = (dg[pl.ds(s, 1), :] == my).astype(jnp.float32)
            csum = jnp.dot(mrow, upper, preferred_element_type=jnp.float32)
            pos = (total + csum - 1.0).astype(jnp.int32)
            sel = ((ii == pos) & (mrow > 0.5)).astype(jnp.bfloat16)
            acc = acc + jnp.dot(sel, xg[s], preferred_element_type=jnp.float32)
            total = total + jnp.sum(mrow)
        out_ref[:, :] = acc.astype(jnp.bfloat16)

    return pl.pallas_call(
        body,
        out_shape=jax.ShapeDtypeStruct((T, D), jnp.bfloat16),
        in_specs=[
            pl.BlockSpec(memory_space=pltpu.VMEM),
            pl.BlockSpec(memory_space=pltpu.VMEM),
        ],
        out_specs=pl.BlockSpec(memory_space=pltpu.VMEM),
        scratch_shapes=[
            pltpu.VMEM((N, T, D), jnp.bfloat16),
            pltpu.VMEM((N, T), jnp.int32),
            pltpu.SemaphoreType.DMA,
            pltpu.SemaphoreType.DMA,
            pltpu.SemaphoreType.DMA,
            pltpu.SemaphoreType.DMA,
        ],
        compiler_params=pltpu.CompilerParams(collective_id=0),
    )(x, dest2)


# device time: 25166 ns/iter; 2.3216x vs baseline; 2.3216x over previous
import jax
import jax.numpy as jnp
from jax import lax
from jax.experimental import pallas as pl
from jax.experimental.pallas import tpu as pltpu

N = 4
T = 1024
D = 512
BLK = 64
SLOT = 384
NS = N * SLOT


def kernel(x, dest):
    dest2 = dest.reshape(1, T)

    def body(x_ref, d_ref, out_ref, xp, stg, dg, sd, rd, sx, rx):
        mx = lax.axis_index("x")
        my = lax.axis_index("y")
        mz = lax.axis_index("z")
        f32 = jnp.float32

        bsem = pltpu.get_barrier_semaphore()
        for dy in range(1, N):
            peer = lax.rem(my + dy, N)
            pl.semaphore_signal(
                bsem, inc=1,
                device_id=(mx, peer, mz),
                device_id_type=pl.DeviceIdType.MESH,
            )
        pl.semaphore_wait(bsem, N - 1)

        dg[pl.ds(my, 1), :] = d_ref[:, :]
        for dy in range(1, N):
            peer = lax.rem(my + dy, N)
            pltpu.make_async_remote_copy(
                src_ref=dg.at[pl.ds(my, 1)],
                dst_ref=dg.at[pl.ds(my, 1)],
                send_sem=sd, recv_sem=rd,
                device_id=(mx, peer, mz),
                device_id_type=pl.DeviceIdType.MESH,
            ).start()

        jj = lax.broadcasted_iota(jnp.int32, (T, T), 1)
        upper = (lax.broadcasted_iota(jnp.int32, (T, T), 0) <= jj).astype(f32)
        d = d_ref[:, :]
        pos = jnp.zeros((1, T), f32)
        cnts = []
        for r in range(N):
            m = (d == r).astype(f32)
            cs = jnp.dot(m, upper, preferred_element_type=f32)
            pos = pos + m * (r * SLOT + cs - 1.0)
            cnts.append(jnp.sum(m))
        qi = lax.broadcasted_iota(jnp.int32, (NS, T), 0)
        perm = (qi == pos.astype(jnp.int32)).astype(jnp.bfloat16)
        xp[:, :] = jnp.dot(
            perm, x_ref[:, :].astype(jnp.bfloat16),
            preferred_element_type=f32,
        ).astype(jnp.bfloat16)

        for _ in range(N - 1):
            dummy_d = pltpu.make_async_remote_copy(
                src_ref=dg.at[pl.ds(0, 1)], dst_ref=dg.at[pl.ds(0, 1)],
                send_sem=sd, recv_sem=rd,
                device_id=(mx, my, mz),
                device_id_type=pl.DeviceIdType.MESH,
            )
            dummy_d.wait_send()
            dummy_d.wait_recv()

        tot_out = f32(0.0)
        for r in range(N):
            nb = jnp.ceil(cnts[r] / BLK)
            tot_out = tot_out + nb

            def send_block(b, carry, r=r):
                dst0 = pl.multiple_of(my * SLOT + b * BLK, BLK)
                pltpu.make_async_remote_copy(
                    src_ref=xp.at[pl.ds(r * SLOT + b * BLK, BLK)],
                    dst_ref=stg.at[pl.ds(dst0, BLK)],
                    send_sem=sx, recv_sem=rx,
                    device_id=(mx, jnp.int32(r), mz),
                    device_id_type=pl.DeviceIdType.MESH,
                ).start()
                return carry

            lax.fori_loop(0, nb.astype(jnp.int32), send_block, 0)

        siota = lax.broadcasted_iota(jnp.int32, (N, 1), 0)
        cin = jnp.sum((dg[:, :] == my).astype(f32), axis=1, keepdims=True)
        qj = lax.broadcasted_iota(jnp.int32, (T, NS), 1)
        oi = lax.broadcasted_iota(jnp.int32, (T, NS), 0)
        viota = lax.broadcasted_iota(jnp.int32, (NS, 1), 0)
        rsel = jnp.zeros((T, NS), jnp.bool_)
        valid = jnp.zeros((NS, 1), jnp.bool_)
        tot_in = f32(0.0)
        off = f32(0.0)
        for s in range(N):
            c_s = jnp.sum(cin * (siota == s).astype(f32))
            tot_in = tot_in + jnp.ceil(c_s / BLK)
            lo = s * SLOT
            ci = c_s.astype(jnp.int32)
            shift = off.astype(jnp.int32) - lo
            rsel = rsel | ((qj >= lo) & (qj < lo + ci) & (oi == qj + shift))
            valid = valid | ((viota >= lo) & (viota < lo + ci))
            off = off + c_s
        rmat = rsel.astype(jnp.bfloat16)

        def wait_send(b, carry):
            pltpu.make_async_remote_copy(
                src_ref=xp.at[pl.ds(0, BLK)],
                dst_ref=stg.at[pl.ds(0, BLK)],
                send_sem=sx, recv_sem=rx,
                device_id=(mx, my, mz),
                device_id_type=pl.DeviceIdType.MESH,
            ).wait_send()
            return carry

        def wait_recv(b, carry):
            pltpu.make_async_remote_copy(
                src_ref=xp.at[pl.ds(0, BLK)],
                dst_ref=stg.at[pl.ds(0, BLK)],
                send_sem=sx, recv_sem=rx,
                device_id=(mx, my, mz),
                device_id_type=pl.DeviceIdType.MESH,
            ).wait_recv()
            return carry

        lax.fori_loop(0, tot_out.astype(jnp.int32), wait_send, 0)
        lax.fori_loop(0, tot_in.astype(jnp.int32), wait_recv, 0)

        stg_vals = jnp.where(valid, stg[:, :], jnp.bfloat16(0.0))
        out_ref[:, :] = jnp.dot(
            rmat, stg_vals, preferred_element_type=f32
        ).astype(jnp.bfloat16)

    return pl.pallas_call(
        body,
        out_shape=jax.ShapeDtypeStruct((T, D), jnp.bfloat16),
        in_specs=[
            pl.BlockSpec(memory_space=pltpu.VMEM),
            pl.BlockSpec(memory_space=pltpu.VMEM),
        ],
        out_specs=pl.BlockSpec(memory_space=pltpu.VMEM),
        scratch_shapes=[
            pltpu.VMEM((NS, D), jnp.bfloat16),
            pltpu.VMEM((NS, D), jnp.bfloat16),
            pltpu.VMEM((N, T), jnp.int32),
            pltpu.SemaphoreType.DMA,
            pltpu.SemaphoreType.DMA,
            pltpu.SemaphoreType.DMA,
            pltpu.SemaphoreType.DMA,
        ],
        compiler_params=pltpu.CompilerParams(collective_id=0),
    )(x, dest2)


# device time: 24560 ns/iter; 2.3789x vs baseline; 1.0247x over previous
import jax
import jax.numpy as jnp
from jax import lax
from jax.experimental import pallas as pl
from jax.experimental.pallas import tpu as pltpu

N = 4
T = 1024
D = 512
BLK = 32
SLOT = 320
NS = N * SLOT


def kernel(x, dest):
    dest2 = dest.reshape(1, T)

    def body(x_ref, d_ref, out_ref, xp, stg, dg, sd, rd, sx, rx):
        mx = lax.axis_index("x")
        my = lax.axis_index("y")
        mz = lax.axis_index("z")
        f32 = jnp.float32

        bsem = pltpu.get_barrier_semaphore()
        for dy in range(1, N):
            peer = lax.rem(my + dy, N)
            pl.semaphore_signal(
                bsem, inc=1,
                device_id=(mx, peer, mz),
                device_id_type=pl.DeviceIdType.MESH,
            )
        pl.semaphore_wait(bsem, N - 1)

        dg[pl.ds(my, 1), :] = d_ref[:, :]
        for dy in range(1, N):
            peer = lax.rem(my + dy, N)
            pltpu.make_async_remote_copy(
                src_ref=dg.at[pl.ds(my, 1)],
                dst_ref=dg.at[pl.ds(my, 1)],
                send_sem=sd, recv_sem=rd,
                device_id=(mx, peer, mz),
                device_id_type=pl.DeviceIdType.MESH,
            ).start()

        jj = lax.broadcasted_iota(jnp.int32, (T, T), 1)
        upper = (lax.broadcasted_iota(jnp.int32, (T, T), 0) <= jj).astype(f32)
        d = d_ref[:, :]
        pos = jnp.zeros((1, T), f32)
        cnts = []
        for r in range(N):
            m = (d == r).astype(f32)
            cs = jnp.dot(m, upper, preferred_element_type=f32)
            pos = pos + m * (r * SLOT + cs - 1.0)
            cnts.append(jnp.sum(m))
        posi = pos.astype(jnp.int32)
        xbf = x_ref[:, :].astype(jnp.bfloat16)

        for _ in range(N - 1):
            dummy_d = pltpu.make_async_remote_copy(
                src_ref=dg.at[pl.ds(0, 1)], dst_ref=dg.at[pl.ds(0, 1)],
                send_sem=sd, recv_sem=rd,
                device_id=(mx, my, mz),
                device_id_type=pl.DeviceIdType.MESH,
            )
            dummy_d.wait_send()
            dummy_d.wait_recv()

        qi = lax.broadcasted_iota(jnp.int32, (SLOT, T), 0)
        tot_out = f32(0.0)
        for r in range(N):
            perm_r = (qi + r * SLOT == posi).astype(jnp.bfloat16)
            xp[pl.ds(r * SLOT, SLOT), :] = jnp.dot(
                perm_r, xbf, preferred_element_type=f32
            ).astype(jnp.bfloat16)
            nb = jnp.ceil(cnts[r] / BLK)
            tot_out = tot_out + nb

            def send_block(b, carry, r=r):
                dst0 = pl.multiple_of(my * SLOT + b * BLK, BLK)
                pltpu.make_async_remote_copy(
                    src_ref=xp.at[pl.ds(r * SLOT + b * BLK, BLK)],
                    dst_ref=stg.at[pl.ds(dst0, BLK)],
                    send_sem=sx, recv_sem=rx,
                    device_id=(mx, jnp.int32(r), mz),
                    device_id_type=pl.DeviceIdType.MESH,
                ).start()
                return carry

            lax.fori_loop(0, nb.astype(jnp.int32), send_block, 0)

        siota = lax.broadcasted_iota(jnp.int32, (N, 1), 0)
        cin = jnp.sum((dg[:, :] == my).astype(f32), axis=1, keepdims=True)
        spos_r = jnp.full((1, NS), -1, jnp.int32)
        valid_c = jnp.zeros((NS, 1), jnp.bool_)
        qrow = lax.broadcasted_iota(jnp.int32, (1, NS), 1)
        qcol = lax.broadcasted_iota(jnp.int32, (NS, 1), 0)
        tot_in = f32(0.0)
        off = f32(0.0)
        for s in range(N):
            c_s = jnp.sum(cin * (siota == s).astype(f32))
            tot_in = tot_in + jnp.ceil(c_s / BLK)
            lo = s * SLOT
            ci = c_s.astype(jnp.int32)
            shift = off.astype(jnp.int32) - lo
            in_seg_r = (qrow >= lo) & (qrow < lo + ci)
            spos_r = jnp.where(in_seg_r, qrow + shift, spos_r)
            valid_c = valid_c | ((qcol >= lo) & (qcol < lo + ci))
            off = off + c_s
        oi = lax.broadcasted_iota(jnp.int32, (T, NS), 0)
        rmat = (oi == spos_r).astype(jnp.bfloat16)

        def wait_recv(b, carry):
            pltpu.make_async_remote_copy(
                src_ref=xp.at[pl.ds(0, BLK)],
                dst_ref=stg.at[pl.ds(0, BLK)],
                send_sem=sx, recv_sem=rx,
                device_id=(mx, my, mz),
                device_id_type=pl.DeviceIdType.MESH,
            ).wait_recv()
            return carry

        lax.fori_loop(0, tot_in.astype(jnp.int32), wait_recv, 0)

        stg_vals = jnp.where(valid_c, stg[:, :], jnp.bfloat16(0.0))
        out_ref[:, :] = jnp.dot(
            rmat, stg_vals, preferred_element_type=f32
        ).astype(jnp.bfloat16)

        def wait_send(b, carry):
            pltpu.make_async_remote_copy(
                src_ref=xp.at[pl.ds(0, BLK)],
                dst_ref=stg.at[pl.ds(0, BLK)],
                send_sem=sx, recv_sem=rx,
                device_id=(mx, my, mz),
                device_id_type=pl.DeviceIdType.MESH,
            ).wait_send()
            return carry

        lax.fori_loop(0, tot_out.astype(jnp.int32), wait_send, 0)

    return pl.pallas_call(
        body,
        out_shape=jax.ShapeDtypeStruct((T, D), jnp.bfloat16),
        in_specs=[
            pl.BlockSpec(memory_space=pltpu.VMEM),
            pl.BlockSpec(memory_space=pltpu.VMEM),
        ],
        out_specs=pl.BlockSpec(memory_space=pltpu.VMEM),
        scratch_shapes=[
            pltpu.VMEM((NS, D), jnp.bfloat16),
            pltpu.VMEM((NS, D), jnp.bfloat16),
            pltpu.VMEM((N, T), jnp.int32),
            pltpu.SemaphoreType.DMA,
            pltpu.SemaphoreType.DMA,
            pltpu.SemaphoreType.DMA,
            pltpu.SemaphoreType.DMA,
        ],
        compiler_params=pltpu.CompilerParams(collective_id=0),
    )(x, dest2)


# device time: 22311 ns/iter; 2.6187x vs baseline; 1.1008x over previous
import jax
import jax.numpy as jnp
from jax import lax
from jax.experimental import pallas as pl
from jax.experimental.pallas import tpu as pltpu

N = 4
T = 1024
D = 512
BLK = 32
SLOT = 320
NS = N * SLOT


def kernel(x, dest):
    dest2 = dest.reshape(1, T)

    def body(x_ref, d_ref, out_ref, xp, stg, dg, sd, rd, sx, rx):
        mx = lax.axis_index("x")
        my = lax.axis_index("y")
        mz = lax.axis_index("z")
        f32 = jnp.float32

        bsem = pltpu.get_barrier_semaphore()
        for dy in range(1, N):
            peer = lax.rem(my + dy, N)
            pl.semaphore_signal(
                bsem, inc=1,
                device_id=(mx, peer, mz),
                device_id_type=pl.DeviceIdType.MESH,
            )
        pl.semaphore_wait(bsem, N - 1)

        dg[pl.ds(my, 1), :] = d_ref[:, :]
        for dy in range(1, N):
            peer = lax.rem(my + dy, N)
            pltpu.make_async_remote_copy(
                src_ref=dg.at[pl.ds(my, 1)],
                dst_ref=dg.at[pl.ds(my, 1)],
                send_sem=sd, recv_sem=rd,
                device_id=(mx, peer, mz),
                device_id_type=pl.DeviceIdType.MESH,
            ).start()

        jj = lax.broadcasted_iota(jnp.int32, (T, T), 1)
        upper = (lax.broadcasted_iota(jnp.int32, (T, T), 0) <= jj).astype(f32)
        d = d_ref[:, :]
        pos = jnp.zeros((1, T), f32)
        cnts = []
        for r in range(N):
            m = (d == r).astype(f32)
            cs = jnp.dot(m, upper, preferred_element_type=f32)
            pos = pos + m * (r * SLOT + cs - 1.0)
            cnts.append(jnp.sum(m))
        posi = pos.astype(jnp.int32)
        xbf = x_ref[:, :].astype(jnp.bfloat16)

        qi = lax.broadcasted_iota(jnp.int32, (SLOT, T), 0)
        tot_out = f32(0.0)
        for r in range(N):
            perm_r = (qi + r * SLOT == posi).astype(jnp.bfloat16)
            xp[pl.ds(r * SLOT, SLOT), :] = jnp.dot(
                perm_r, xbf, preferred_element_type=f32
            ).astype(jnp.bfloat16)
            nb = jnp.ceil(cnts[r] / BLK)
            tot_out = tot_out + nb

            def send_block(b, carry, r=r):
                dst0 = pl.multiple_of(my * SLOT + b * BLK, BLK)
                pltpu.make_async_remote_copy(
                    src_ref=xp.at[pl.ds(r * SLOT + b * BLK, BLK)],
                    dst_ref=stg.at[pl.ds(dst0, BLK)],
                    send_sem=sx, recv_sem=rx,
                    device_id=(mx, jnp.int32(r), mz),
                    device_id_type=pl.DeviceIdType.MESH,
                ).start()
                return carry

            lax.fori_loop(0, nb.astype(jnp.int32), send_block, 0)

        for _ in range(N - 1):
            dummy_d = pltpu.make_async_remote_copy(
                src_ref=dg.at[pl.ds(0, 1)], dst_ref=dg.at[pl.ds(0, 1)],
                send_sem=sd, recv_sem=rd,
                device_id=(mx, my, mz),
                device_id_type=pl.DeviceIdType.MESH,
            )
            dummy_d.wait_send()
            dummy_d.wait_recv()

        siota = lax.broadcasted_iota(jnp.int32, (N, 1), 0)
        cin = jnp.sum((dg[:, :] == my).astype(f32), axis=1, keepdims=True)
        spos_r = jnp.full((1, NS), -1, jnp.int32)
        valid_c = jnp.zeros((NS, 1), jnp.bool_)
        qrow = lax.broadcasted_iota(jnp.int32, (1, NS), 1)
        qcol = lax.broadcasted_iota(jnp.int32, (NS, 1), 0)
        tot_in = f32(0.0)
        off = f32(0.0)
        for s in range(N):
            c_s = jnp.sum(cin * (siota == s).astype(f32))
            tot_in = tot_in + jnp.ceil(c_s / BLK)
            lo = s * SLOT
            ci = c_s.astype(jnp.int32)
            shift = off.astype(jnp.int32) - lo
            in_seg_r = (qrow >= lo) & (qrow < lo + ci)
            spos_r = jnp.where(in_seg_r, qrow + shift, spos_r)
            valid_c = valid_c | ((qcol >= lo) & (qcol < lo + ci))
            off = off + c_s
        oi = lax.broadcasted_iota(jnp.int32, (T, NS), 0)
        rmat = (oi == spos_r).astype(jnp.bfloat16)

        def wait_recv(b, carry):
            pltpu.make_async_remote_copy(
                src_ref=xp.at[pl.ds(0, BLK)],
                dst_ref=stg.at[pl.ds(0, BLK)],
                send_sem=sx, recv_sem=rx,
                device_id=(mx, my, mz),
                device_id_type=pl.DeviceIdType.MESH,
            ).wait_recv()
            return carry

        lax.fori_loop(0, tot_in.astype(jnp.int32), wait_recv, 0)

        stg_vals = jnp.where(valid_c, stg[:, :], jnp.bfloat16(0.0))
        out_ref[:, :] = jnp.dot(
            rmat, stg_vals, preferred_element_type=f32
        ).astype(jnp.bfloat16)

        def wait_send(b, carry):
            pltpu.make_async_remote_copy(
                src_ref=xp.at[pl.ds(0, BLK)],
                dst_ref=stg.at[pl.ds(0, BLK)],
                send_sem=sx, recv_sem=rx,
                device_id=(mx, my, mz),
                device_id_type=pl.DeviceIdType.MESH,
            ).wait_send()
            return carry

        lax.fori_loop(0, tot_out.astype(jnp.int32), wait_send, 0)

    return pl.pallas_call(
        body,
        out_shape=jax.ShapeDtypeStruct((T, D), jnp.bfloat16),
        in_specs=[
            pl.BlockSpec(memory_space=pltpu.VMEM),
            pl.BlockSpec(memory_space=pltpu.VMEM),
        ],
        out_specs=pl.BlockSpec(memory_space=pltpu.VMEM),
        scratch_shapes=[
            pltpu.VMEM((NS, D), jnp.bfloat16),
            pltpu.VMEM((NS, D), jnp.bfloat16),
            pltpu.VMEM((N, T), jnp.int32),
            pltpu.SemaphoreType.DMA,
            pltpu.SemaphoreType.DMA,
            pltpu.SemaphoreType.DMA,
            pltpu.SemaphoreType.DMA,
        ],
        compiler_params=pltpu.CompilerParams(collective_id=0),
    )(x, dest2)
